# Initial kernel scaffold; baseline (speedup 1.0000x reference)
import jax
import jax.numpy as jnp
from jax.experimental import pallas as pl


def kernel(x, z, edge_index, edge_attr, batch, params):
    raise NotImplementedError("write your pallas kernel here")


# speedup vs baseline: 3.5181x; 3.5181x over previous
import functools

import jax
import jax.numpy as jnp
from jax import lax
from jax.experimental import pallas as pl
from jax.experimental.pallas import tpu as pltpu
from jax.experimental.pallas import tpu_sc as plsc

NC = 2
NS = 16
NW = NC * NS
LANES = 16


def _sc_mesh():
    return plsc.VectorSubcoreMesh(
        core_axis_name="c", subcore_axis_name="s",
        num_cores=NC, num_subcores=NS)


def _make_sc_gather(V, D, B):
    assert D % LANES == 0 and B % (8 * NW) == 0
    bpw = B // NW
    CH = 64
    assert bpw % CH == 0
    nch = bpw // CH

    @functools.partial(
        pl.kernel,
        out_type=jax.ShapeDtypeStruct((B, D), jnp.float32),
        mesh=_sc_mesh(),
        scratch_types=[
            pltpu.VMEM((CH,), jnp.int32),
            pltpu.VMEM((bpw, D), jnp.float32),
            pltpu.SemaphoreType.DMA,
        ],
    )
    def k(table_hbm, idx_hbm, out_hbm, idx_v, rows_v, sem):
        wid = lax.axis_index("s") * NC + lax.axis_index("c")
        base = wid * bpw
        for j in range(nch):
            pltpu.sync_copy(idx_hbm.at[pl.ds(base + j * CH, CH)], idx_v)
            pltpu.async_copy(
                table_hbm.at[idx_v], rows_v.at[pl.ds(j * CH, CH)], sem
            ).wait()
        pltpu.sync_copy(rows_v, out_hbm.at[pl.ds(base, bpw)])

    return k


def _make_sc_aggregate(N, E, D):
    C = 128
    assert E % C == 0 and D % LANES == 0
    nch_total = E // C
    per_w = -(-nch_total // NW)
    ZB = 80
    assert N % ZB == 0 and ZB <= C
    n_zch = N // ZB
    zper = -(-n_zch // NS)

    @functools.partial(
        pl.kernel,
        out_type=jax.ShapeDtypeStruct((NC, N, D), jnp.float32),
        mesh=_sc_mesh(),
        scratch_types=[
            pltpu.VMEM((C,), jnp.int32),
            pltpu.VMEM((C,), jnp.int32),
            pltpu.VMEM((C, D), jnp.float32),
            pltpu.VMEM((C, D), jnp.float32),
            pltpu.VMEM_SHARED((N, D), jnp.float32),
            pltpu.SemaphoreType.DMA,
            pltpu.SemaphoreType.DMA,
        ],
    )
    def k(h_hbm, e_hbm, src_hbm, dst_hbm, out_hbm,
          sidx, didx, hrow, erow, aggr, sem0, sem1):
        cid = lax.axis_index("c")
        tid = lax.axis_index("s")
        wid = tid * NC + cid

        zv = jnp.zeros((LANES,), jnp.float32)

        def zbody(r, carry):
            for j in range(D // LANES):
                hrow[r, pl.ds(j * LANES, LANES)] = zv
            return carry

        lax.fori_loop(0, C, zbody, 0)
        for kk in range(zper):
            zc = kk * NS + tid

            @pl.when(zc < n_zch)
            def _():
                pltpu.sync_copy(hrow.at[pl.ds(0, ZB)],
                                aggr.at[pl.ds(zc * ZB, ZB)])

        plsc.subcore_barrier()

        def chunk_body(i, carry):
            ch = i * NW + wid

            @pl.when(ch < nch_total)
            def _():
                base = ch * C
                pltpu.sync_copy(src_hbm.at[pl.ds(base, C)], sidx)
                pltpu.sync_copy(dst_hbm.at[pl.ds(base, C)], didx)
                cp_h = pltpu.async_copy(h_hbm.at[sidx], hrow, sem0)
                cp_e = pltpu.async_copy(e_hbm.at[pl.ds(base, C)], erow, sem1)
                cp_h.wait()
                cp_e.wait()

                def rbody(r, cc):
                    for j in range(D // LANES):
                        sl = pl.ds(j * LANES, LANES)
                        erow[r, sl] = jnp.maximum(hrow[r, sl] + erow[r, sl],
                                                  0.0)
                    return cc

                lax.fori_loop(0, C, rbody, 0)
                pltpu.sync_copy(erow, aggr.at[didx], add=True)

            return carry

        lax.fori_loop(0, per_w, chunk_body, 0)
        plsc.subcore_barrier()

        for kk in range(zper):
            zc = kk * NS + tid

            @pl.when(zc < n_zch)
            def _():
                r0 = zc * ZB
                pltpu.sync_copy(aggr.at[pl.ds(r0, ZB)], hrow.at[pl.ds(0, ZB)])
                pltpu.sync_copy(hrow.at[pl.ds(0, ZB)],
                                out_hbm.at[cid, pl.ds(r0, ZB)])

    return k


def _make_sc_pool(B, DP, G):
    NP = 3
    assert B % (8 * NW) == 0 and G % NS == 0 and DP % LANES == 0
    bpw = B // NW
    CH = 64
    assert bpw % CH == 0
    nch = bpw // CH
    gpt = G // NS
    assert gpt % 8 == 0 and gpt <= CH

    @functools.partial(
        pl.kernel,
        out_type=jax.ShapeDtypeStruct((NC, NP, G, DP), jnp.float32),
        mesh=_sc_mesh(),
        scratch_types=[
            pltpu.VMEM((CH,), jnp.int32),
            pltpu.VMEM((CH, DP), jnp.float32),
            pltpu.VMEM_SHARED((NP, G, DP), jnp.float32),
            pltpu.SemaphoreType.DMA,
        ],
    )
    def k(h1_hbm, h2_hbm, h3_hbm, bidx_hbm, out_hbm, idx_v, rows_v, pool,
          sem):
        cid = lax.axis_index("c")
        tid = lax.axis_index("s")
        wid = tid * NC + cid
        hs = (h1_hbm, h2_hbm, h3_hbm)

        zv = jnp.zeros((LANES,), jnp.float32)

        def zbody(r, carry):
            for j in range(DP // LANES):
                rows_v[r, pl.ds(j * LANES, LANES)] = zv
            return carry

        lax.fori_loop(0, CH, zbody, 0)
        for p in range(NP):
            pltpu.sync_copy(rows_v.at[pl.ds(0, gpt)],
                            pool.at[p, pl.ds(tid * gpt, gpt)])
        plsc.subcore_barrier()

        base = wid * bpw
        for j in range(nch):
            b = base + j * CH
            pltpu.sync_copy(bidx_hbm.at[pl.ds(b, CH)], idx_v)
            for p in range(NP):
                pltpu.async_copy(hs[p].at[pl.ds(b, CH)], rows_v, sem).wait()
                pltpu.sync_copy(rows_v, pool.at[p].at[idx_v], add=True)

        plsc.subcore_barrier()
        for p in range(NP):
            pltpu.sync_copy(pool.at[p, pl.ds(tid * gpt, gpt)],
                            rows_v.at[pl.ds(0, gpt)])
            pltpu.sync_copy(rows_v.at[pl.ds(0, gpt)],
                            out_hbm.at[cid, p, pl.ds(tid * gpt, gpt)])

    return k


def _tc_matmul_bias_relu(xin, w, b, block_rows, relu=True):
    n, kdim = xin.shape
    kd2, m = w.shape
    assert kdim == kd2 and n % block_rows == 0

    def body(x_ref, w_ref, b_ref, o_ref):
        y = jnp.dot(x_ref[...], w_ref[...],
                    preferred_element_type=jnp.float32) + b_ref[...]
        if relu:
            y = jnp.maximum(y, 0.0)
        o_ref[...] = y

    return pl.pallas_call(
        body,
        grid=(n // block_rows,),
        in_specs=[
            pl.BlockSpec((block_rows, kdim), lambda i: (i, 0)),
            pl.BlockSpec((kd2, m), lambda i: (0, 0)),
            pl.BlockSpec((1, m), lambda i: (0, 0)),
        ],
        out_specs=pl.BlockSpec((block_rows, m), lambda i: (i, 0)),
        out_shape=jax.ShapeDtypeStruct((n, m), jnp.float32),
    )(xin, w, b)


def _tc_encode(ze, x16, wx, b, block_rows):
    n, d = ze.shape
    kdim = x16.shape[1]

    def body(ze_ref, x_ref, w_ref, b_ref, o_ref):
        o_ref[...] = ze_ref[...] + jnp.dot(
            x_ref[...], w_ref[...],
            preferred_element_type=jnp.float32) + b_ref[...]

    return pl.pallas_call(
        body,
        grid=(n // block_rows,),
        in_specs=[
            pl.BlockSpec((block_rows, d), lambda i: (i, 0)),
            pl.BlockSpec((block_rows, kdim), lambda i: (i, 0)),
            pl.BlockSpec((kdim, d), lambda i: (0, 0)),
            pl.BlockSpec((1, d), lambda i: (0, 0)),
        ],
        out_specs=pl.BlockSpec((block_rows, d), lambda i: (i, 0)),
        out_shape=jax.ShapeDtypeStruct((n, d), jnp.float32),
    )(ze, x16, wx, b)


def _tc_mlp_layer(h, part, w1, b1, w2, b2, block_rows):
    n, d = h.shape

    def body(h_ref, a0_ref, a1_ref, w1_ref, b1_ref, w2_ref, b2_ref, o_ref):
        y = h_ref[...] + a0_ref[0] + a1_ref[0]
        t = jnp.maximum(
            jnp.dot(y, w1_ref[...], preferred_element_type=jnp.float32)
            + b1_ref[...], 0.0)
        o_ref[...] = jnp.maximum(
            jnp.dot(t, w2_ref[...], preferred_element_type=jnp.float32)
            + b2_ref[...], 0.0)

    return pl.pallas_call(
        body,
        grid=(n // block_rows,),
        in_specs=[
            pl.BlockSpec((block_rows, d), lambda i: (i, 0)),
            pl.BlockSpec((1, block_rows, d), lambda i: (0, i, 0)),
            pl.BlockSpec((1, block_rows, d), lambda i: (1, i, 0)),
            pl.BlockSpec((d, d), lambda i: (0, 0)),
            pl.BlockSpec((1, d), lambda i: (0, 0)),
            pl.BlockSpec((d, d), lambda i: (0, 0)),
            pl.BlockSpec((1, d), lambda i: (0, 0)),
        ],
        out_specs=pl.BlockSpec((block_rows, d), lambda i: (i, 0)),
        out_shape=jax.ShapeDtypeStruct((n, d), jnp.float32),
    )(h, part, part, w1, b1, w2, b2)


def _tc_head(pools, wl3, bl, G):
    _, NP, gp, dp = pools.shape
    m = wl3.shape[2]

    def body(p0_ref, p1_ref, w_ref, b_ref, o_ref):
        acc = b_ref[...]
        for p in range(NP):
            pp = p0_ref[0, p] + p1_ref[0, p]
            acc = acc + jnp.dot(pp, w_ref[p],
                                preferred_element_type=jnp.float32)
        o_ref[...] = jnp.maximum(acc, 0.0)

    return pl.pallas_call(
        body,
        grid=(1,),
        in_specs=[
            pl.BlockSpec((1, NP, G, dp), lambda i: (0, 0, 0, 0)),
            pl.BlockSpec((1, NP, G, dp), lambda i: (1, 0, 0, 0)),
            pl.BlockSpec((NP, dp, m), lambda i: (0, 0, 0)),
            pl.BlockSpec((1, m), lambda i: (0, 0)),
        ],
        out_specs=pl.BlockSpec((G, m), lambda i: (0, 0)),
        out_shape=jax.ShapeDtypeStruct((G, m), jnp.float32),
    )(pools, pools, wl3, bl)


def _fold_bn(wmat, bvec, bn, eps=1e-5):
    gamma, beta, rm, rv = bn
    s = gamma / jnp.sqrt(rv + eps)
    return wmat * s[None, :], (bvec - rm) * s + beta


def kernel(x, z, edge_index, edge_attr, batch, params):
    N = x.shape[0]
    E = edge_index.shape[1]
    HID = params["edge_emb"].shape[1]
    G = 512
    NPAD = ((N + 8 * NW - 1) // (8 * NW)) * (8 * NW)
    GPAD = G + 128

    ztab = params["z_emb"]
    zw = ztab.shape[1]
    wi, bi = params["init_proj"]
    ztab_proj = ztab @ wi[:zw]
    z_pad = jnp.concatenate(
        [z.astype(jnp.int32), jnp.zeros((NPAD - N,), jnp.int32)])

    xw = x.shape[1]
    XPAD = 16
    x16 = jnp.concatenate([x, jnp.zeros((N, XPAD - xw), jnp.float32)], axis=1)
    wx = jnp.concatenate(
        [wi[zw:], jnp.zeros((XPAD - xw, HID), jnp.float32)], axis=0)

    src = edge_index[0].astype(jnp.int32)
    dst = edge_index[1].astype(jnp.int32)

    ze = _make_sc_gather(ztab_proj.shape[0], HID, NPAD)(ztab_proj, z_pad)[:N]

    h = _tc_encode(ze, x16, wx, bi.reshape(1, HID), block_rows=2000)
    e = _tc_matmul_bias_relu(edge_attr, params["edge_emb"],
                             jnp.zeros((1, HID), jnp.float32),
                             block_rows=6400, relu=False)

    agg = _make_sc_aggregate(N, E, HID)
    xs = []
    for layer in params["layers"]:
        part = agg(h, e, src, dst)
        w1f, b1f = _fold_bn(layer["W1"], layer["b1"], layer["bn1"])
        w2f, b2f = _fold_bn(layer["W2"], layer["b2"], layer["bn2"])
        h = _tc_mlp_layer(h, part, w1f, b1f.reshape(1, HID),
                          w2f, b2f.reshape(1, HID), block_rows=2000)
        xs.append(h)

    pad = jnp.zeros((NPAD - N, HID), jnp.float32)
    hp = [jnp.concatenate([hh, pad], axis=0) for hh in xs]
    dump = G + (jnp.arange(NPAD - N, dtype=jnp.int32) % (GPAD - G))
    bidx = jnp.concatenate([batch.astype(jnp.int32), dump])
    pools = _make_sc_pool(NPAD, HID, GPAD)(hp[0], hp[1], hp[2], bidx)

    wl, bl = params["lin1"]
    wl3 = wl.reshape(3, HID, HID)
    return _tc_head(pools[:, :, :G], wl3, bl.reshape(1, HID), G)


# speedup vs baseline: 5.5715x; 1.5837x over previous
import functools

import jax
import jax.numpy as jnp
from jax import lax
from jax.experimental import pallas as pl
from jax.experimental.pallas import tpu as pltpu
from jax.experimental.pallas import tpu_sc as plsc

NC = 2
NS = 16
NW = NC * NS
LANES = 16


def _sc_mesh():
    return plsc.VectorSubcoreMesh(
        core_axis_name="c", subcore_axis_name="s",
        num_cores=NC, num_subcores=NS)


def _make_sc_gather(V, D, B):
    assert D % LANES == 0 and B % (8 * NW) == 0
    bpw = B // NW
    CH = 64
    assert bpw % CH == 0
    nch = bpw // CH

    @functools.partial(
        pl.kernel,
        out_type=jax.ShapeDtypeStruct((B, D), jnp.float32),
        mesh=_sc_mesh(),
        scratch_types=[
            pltpu.VMEM((CH,), jnp.int32),
            pltpu.VMEM((bpw, D), jnp.float32),
            pltpu.SemaphoreType.DMA,
        ],
    )
    def k(table_hbm, idx_hbm, out_hbm, idx_v, rows_v, sem):
        wid = lax.axis_index("s") * NC + lax.axis_index("c")
        base = wid * bpw
        for j in range(nch):
            pltpu.sync_copy(idx_hbm.at[pl.ds(base + j * CH, CH)], idx_v)
            pltpu.async_copy(
                table_hbm.at[idx_v], rows_v.at[pl.ds(j * CH, CH)], sem
            ).wait()
        pltpu.sync_copy(rows_v, out_hbm.at[pl.ds(base, bpw)])

    return k


def _make_sc_aggregate(N, NAGG, E, D):
    C = 64
    assert E % (C * NW) == 0 and D % LANES == 0
    per_w = E // (C * NW)
    assert per_w % 4 == 0 and per_w >= 8
    ZB = 40
    assert NAGG % ZB == 0 and N % ZB == 0 and ZB <= C
    n_zch = NAGG // ZB
    n_ozch = N // ZB
    zper = -(-n_zch // NS)

    @functools.partial(
        pl.kernel,
        out_type=jax.ShapeDtypeStruct((NC, N, D), jnp.float32),
        mesh=_sc_mesh(),
        scratch_types=[
            pltpu.VMEM((4, C), jnp.int32),
            pltpu.VMEM((4, C), jnp.int32),
            pltpu.VMEM((C, D), jnp.float32),
            pltpu.VMEM((C, D), jnp.float32),
            pltpu.VMEM((C, D), jnp.float32),
            pltpu.VMEM((C, D), jnp.float32),
            pltpu.VMEM((C, D), jnp.float32),
            pltpu.VMEM((C, D), jnp.float32),
            pltpu.VMEM_SHARED((NAGG, D), jnp.float32),
        ] + [pltpu.SemaphoreType.DMA] * 12,
    )
    def k(h_hbm, e_hbm, src_hbm, dst_hbm, out_hbm,
          sidx, didx, hrow0, hrow1, erow0, erow1, msg0, msg1, aggr,
          si0, si1, si2, si3, ss0, ss1, ss2, ss3, sh0, sh1, se0, se1):
        cid = lax.axis_index("c")
        tid = lax.axis_index("s")
        wid = tid * NC + cid
        hrow = (hrow0, hrow1)
        erow = (erow0, erow1)
        msg = (msg0, msg1)
        sem_i = (si0, si1, si2, si3)
        sem_s = (ss0, ss1, ss2, ss3)
        sem_h = (sh0, sh1)
        sem_e = (se0, se1)

        zv = jnp.zeros((LANES,), jnp.float32)

        def zbody(r, carry):
            for j in range(D // LANES):
                msg0[r, pl.ds(j * LANES, LANES)] = zv
            return carry

        lax.fori_loop(0, C, zbody, 0)
        for kk in range(zper):
            zc = kk * NS + tid

            @pl.when(zc < n_zch)
            def _():
                pltpu.sync_copy(msg0.at[pl.ds(0, ZB)],
                                aggr.at[pl.ds(zc * ZB, ZB)])

        plsc.subcore_barrier()

        def chunk_base(i):
            return (i * NW + wid) * C

        def issue_idx(i, slot):
            b = chunk_base(i)
            pltpu.async_copy(src_hbm.at[pl.ds(b, C)], sidx.at[slot],
                             sem_i[slot])
            pltpu.async_copy(dst_hbm.at[pl.ds(b, C)], didx.at[slot],
                             sem_i[slot])

        def wait_idx(slot):
            pltpu.make_async_copy(src_hbm.at[pl.ds(0, C)], sidx.at[slot],
                                  sem_i[slot]).wait()
            pltpu.make_async_copy(dst_hbm.at[pl.ds(0, C)], didx.at[slot],
                                  sem_i[slot]).wait()

        def issue_data(i, b, slot):
            pltpu.async_copy(h_hbm.at[sidx.at[slot]], hrow[b], sem_h[b])
            pltpu.async_copy(e_hbm.at[pl.ds(chunk_base(i), C)], erow[b],
                             sem_e[b])

        def wait_data(b):
            pltpu.make_async_copy(e_hbm.at[pl.ds(0, C)], hrow[b],
                                  sem_h[b]).wait()
            pltpu.make_async_copy(e_hbm.at[pl.ds(0, C)], erow[b],
                                  sem_e[b]).wait()

        def issue_scatter(b, slot):
            pltpu.make_async_copy(msg[b], aggr.at[didx.at[slot]],
                                  sem_s[slot]).start(add=True)

        def wait_scatter(b, slot):
            pltpu.make_async_copy(msg[b], aggr.at[pl.ds(0, C)],
                                  sem_s[slot]).wait()

        def compute(b):
            def rbody(r, cc):
                for j in range(D // LANES):
                    sl = pl.ds(j * LANES, LANES)
                    msg[b][r, sl] = jnp.maximum(
                        hrow[b][r, sl] + erow[b][r, sl], 0.0)
                return cc

            lax.fori_loop(0, C, rbody, 0)

        _PIPELINED = True
        if not _PIPELINED:
            def chunk_sync(i, carry):
                b0 = chunk_base(i)
                d1 = pltpu.async_copy(src_hbm.at[pl.ds(b0, C)], sidx.at[0],
                                      si0)
                d2 = pltpu.async_copy(dst_hbm.at[pl.ds(b0, C)], didx.at[0],
                                      si1)
                d1.wait()
                d2.wait()
                dh = pltpu.async_copy(h_hbm.at[sidx.at[0]], hrow0, sh0)
                de = pltpu.async_copy(e_hbm.at[pl.ds(b0, C)], erow0, se0)
                dh.wait()
                de.wait()
                compute(0)
                dsc = pltpu.make_async_copy(msg0, aggr.at[didx.at[0]], ss0)
                dsc.start(add=True)
                dsc.wait()
                return carry

            lax.fori_loop(0, per_w, chunk_sync, 0)
            plsc.subcore_barrier()
            for kk in range(zper):
                zc = kk * NS + tid

                @pl.when(zc < n_ozch)
                def _():
                    r0 = zc * ZB
                    pltpu.sync_copy(aggr.at[pl.ds(r0, ZB)],
                                    msg0.at[pl.ds(0, ZB)])
                    pltpu.sync_copy(msg0.at[pl.ds(0, ZB)],
                                    out_hbm.at[cid, pl.ds(r0, ZB)])
            return

        issue_idx(0, 0)
        issue_idx(1, 1)
        wait_idx(0)
        issue_data(0, 0, 0)

        def outer(kq, carry):
            for ph in range(4):
                i = kq * 4 + ph
                b = ph % 2
                s = ph
                s1 = (ph + 1) % 4
                s2 = (ph + 2) % 4

                @pl.when(i >= 2)
                def _():
                    wait_scatter(b, s2)

                @pl.when(i + 2 < per_w)
                def _():
                    issue_idx(i + 2, s2)

                @pl.when(i + 1 < per_w)
                def _():
                    wait_idx(s1)
                    issue_data(i + 1, 1 - b, s1)

                wait_data(b)
                compute(b)
                issue_scatter(b, s)
            return carry

        lax.fori_loop(0, per_w // 4, outer, 0)
        wait_scatter(0, (per_w - 2) % 4)
        wait_scatter(1, (per_w - 1) % 4)
        plsc.subcore_barrier()

        for kk in range(zper):
            zc = kk * NS + tid

            @pl.when(zc < n_ozch)
            def _():
                r0 = zc * ZB
                pltpu.sync_copy(aggr.at[pl.ds(r0, ZB)], msg0.at[pl.ds(0, ZB)])
                pltpu.sync_copy(msg0.at[pl.ds(0, ZB)],
                                out_hbm.at[cid, pl.ds(r0, ZB)])

    return k


def _make_sc_pool(B, DP, G):
    NP = 3
    assert B % (8 * NW) == 0 and G % NS == 0 and DP % LANES == 0
    bpw = B // NW
    CH = 64
    assert bpw % CH == 0
    nch = bpw // CH
    gpt = G // NS
    assert gpt % 8 == 0 and gpt <= CH

    @functools.partial(
        pl.kernel,
        out_type=jax.ShapeDtypeStruct((NC, NP, G, DP), jnp.float32),
        mesh=_sc_mesh(),
        scratch_types=[
            pltpu.VMEM((CH,), jnp.int32),
            pltpu.VMEM((CH, DP), jnp.float32),
            pltpu.VMEM_SHARED((NP, G, DP), jnp.float32),
            pltpu.SemaphoreType.DMA,
        ],
    )
    def k(h1_hbm, h2_hbm, h3_hbm, bidx_hbm, out_hbm, idx_v, rows_v, pool,
          sem):
        cid = lax.axis_index("c")
        tid = lax.axis_index("s")
        wid = tid * NC + cid
        hs = (h1_hbm, h2_hbm, h3_hbm)

        zv = jnp.zeros((LANES,), jnp.float32)

        def zbody(r, carry):
            for j in range(DP // LANES):
                rows_v[r, pl.ds(j * LANES, LANES)] = zv
            return carry

        lax.fori_loop(0, CH, zbody, 0)
        for p in range(NP):
            pltpu.sync_copy(rows_v.at[pl.ds(0, gpt)],
                            pool.at[p, pl.ds(tid * gpt, gpt)])
        plsc.subcore_barrier()

        base = wid * bpw
        for j in range(nch):
            b = base + j * CH
            pltpu.sync_copy(bidx_hbm.at[pl.ds(b, CH)], idx_v)
            for p in range(NP):
                pltpu.async_copy(hs[p].at[pl.ds(b, CH)], rows_v, sem).wait()
                pltpu.sync_copy(rows_v, pool.at[p].at[idx_v], add=True)

        plsc.subcore_barrier()
        for p in range(NP):
            pltpu.sync_copy(pool.at[p, pl.ds(tid * gpt, gpt)],
                            rows_v.at[pl.ds(0, gpt)])
            pltpu.sync_copy(rows_v.at[pl.ds(0, gpt)],
                            out_hbm.at[cid, p, pl.ds(tid * gpt, gpt)])

    return k


def _tc_matmul_bias_relu(xin, w, b, block_rows, relu=True):
    n, kdim = xin.shape
    kd2, m = w.shape
    assert kdim == kd2 and n % block_rows == 0

    def body(x_ref, w_ref, b_ref, o_ref):
        y = jnp.dot(x_ref[...], w_ref[...],
                    preferred_element_type=jnp.float32) + b_ref[...]
        if relu:
            y = jnp.maximum(y, 0.0)
        o_ref[...] = y

    return pl.pallas_call(
        body,
        grid=(n // block_rows,),
        in_specs=[
            pl.BlockSpec((block_rows, kdim), lambda i: (i, 0)),
            pl.BlockSpec((kd2, m), lambda i: (0, 0)),
            pl.BlockSpec((1, m), lambda i: (0, 0)),
        ],
        out_specs=pl.BlockSpec((block_rows, m), lambda i: (i, 0)),
        out_shape=jax.ShapeDtypeStruct((n, m), jnp.float32),
    )(xin, w, b)


def _tc_encode(ze, x16, wx, b, block_rows):
    n, d = ze.shape
    kdim = x16.shape[1]

    def body(ze_ref, x_ref, w_ref, b_ref, o_ref):
        o_ref[...] = ze_ref[...] + jnp.dot(
            x_ref[...], w_ref[...],
            preferred_element_type=jnp.float32) + b_ref[...]

    return pl.pallas_call(
        body,
        grid=(n // block_rows,),
        in_specs=[
            pl.BlockSpec((block_rows, d), lambda i: (i, 0)),
            pl.BlockSpec((block_rows, kdim), lambda i: (i, 0)),
            pl.BlockSpec((kdim, d), lambda i: (0, 0)),
            pl.BlockSpec((1, d), lambda i: (0, 0)),
        ],
        out_specs=pl.BlockSpec((block_rows, d), lambda i: (i, 0)),
        out_shape=jax.ShapeDtypeStruct((n, d), jnp.float32),
    )(ze, x16, wx, b)


def _tc_mlp_layer(h, part, w1, b1, w2, b2, block_rows):
    n, d = h.shape

    def body(h_ref, a0_ref, a1_ref, w1_ref, b1_ref, w2_ref, b2_ref, o_ref):
        y = h_ref[...] + a0_ref[0] + a1_ref[0]
        t = jnp.maximum(
            jnp.dot(y, w1_ref[...], preferred_element_type=jnp.float32)
            + b1_ref[...], 0.0)
        o_ref[...] = jnp.maximum(
            jnp.dot(t, w2_ref[...], preferred_element_type=jnp.float32)
            + b2_ref[...], 0.0)

    return pl.pallas_call(
        body,
        grid=(n // block_rows,),
        in_specs=[
            pl.BlockSpec((block_rows, d), lambda i: (i, 0)),
            pl.BlockSpec((1, block_rows, d), lambda i: (0, i, 0)),
            pl.BlockSpec((1, block_rows, d), lambda i: (1, i, 0)),
            pl.BlockSpec((d, d), lambda i: (0, 0)),
            pl.BlockSpec((1, d), lambda i: (0, 0)),
            pl.BlockSpec((d, d), lambda i: (0, 0)),
            pl.BlockSpec((1, d), lambda i: (0, 0)),
        ],
        out_specs=pl.BlockSpec((block_rows, d), lambda i: (i, 0)),
        out_shape=jax.ShapeDtypeStruct((n, d), jnp.float32),
    )(h, part, part, w1, b1, w2, b2)


def _tc_head(pools, wl3, bl, G):
    _, NP, gp, dp = pools.shape
    m = wl3.shape[2]

    def body(p0_ref, p1_ref, w_ref, b_ref, o_ref):
        acc = b_ref[...]
        for p in range(NP):
            pp = p0_ref[0, p] + p1_ref[0, p]
            acc = acc + jnp.dot(pp, w_ref[p],
                                preferred_element_type=jnp.float32)
        o_ref[...] = jnp.maximum(acc, 0.0)

    return pl.pallas_call(
        body,
        grid=(1,),
        in_specs=[
            pl.BlockSpec((1, NP, G, dp), lambda i: (0, 0, 0, 0)),
            pl.BlockSpec((1, NP, G, dp), lambda i: (1, 0, 0, 0)),
            pl.BlockSpec((NP, dp, m), lambda i: (0, 0, 0)),
            pl.BlockSpec((1, m), lambda i: (0, 0)),
        ],
        out_specs=pl.BlockSpec((G, m), lambda i: (0, 0)),
        out_shape=jax.ShapeDtypeStruct((G, m), jnp.float32),
    )(pools, pools, wl3, bl)


def _fold_bn(wmat, bvec, bn, eps=1e-5):
    gamma, beta, rm, rv = bn
    s = gamma / jnp.sqrt(rv + eps)
    return wmat * s[None, :], (bvec - rm) * s + beta


def kernel(x, z, edge_index, edge_attr, batch, params):
    N = x.shape[0]
    E = edge_index.shape[1]
    HID = params["edge_emb"].shape[1]
    G = 512
    NPAD = ((N + 8 * NW - 1) // (8 * NW)) * (8 * NW)
    GPAD = G + 128

    ztab = params["z_emb"]
    zw = ztab.shape[1]
    wi, bi = params["init_proj"]
    ztab_proj = ztab @ wi[:zw]
    z_pad = jnp.concatenate(
        [z.astype(jnp.int32), jnp.zeros((NPAD - N,), jnp.int32)])

    xw = x.shape[1]
    XPAD = 16
    x16 = jnp.concatenate([x, jnp.zeros((N, XPAD - xw), jnp.float32)], axis=1)
    wx = jnp.concatenate(
        [wi[zw:], jnp.zeros((XPAD - xw, HID), jnp.float32)], axis=0)

    EPAD = 80 * NW * 128
    NAGG = N + 80
    pad_n = EPAD - E
    pidx = jnp.arange(pad_n, dtype=jnp.int32)
    src = jnp.concatenate([edge_index[0].astype(jnp.int32), pidx % N])
    dst = jnp.concatenate([edge_index[1].astype(jnp.int32),
                           N + pidx % (NAGG - N)])
    ea_pad = jnp.concatenate(
        [edge_attr, jnp.zeros((pad_n, edge_attr.shape[1]), jnp.float32)],
        axis=0)

    ze = _make_sc_gather(ztab_proj.shape[0], HID, NPAD)(ztab_proj, z_pad)[:N]

    h = _tc_encode(ze, x16, wx, bi.reshape(1, HID), block_rows=2000)
    e = _tc_matmul_bias_relu(ea_pad, params["edge_emb"],
                             jnp.zeros((1, HID), jnp.float32),
                             block_rows=5120, relu=False)

    agg = _make_sc_aggregate(N, NAGG, EPAD, HID)
    xs = []
    for layer in params["layers"]:
        part = agg(h, e, src, dst)
        w1f, b1f = _fold_bn(layer["W1"], layer["b1"], layer["bn1"])
        w2f, b2f = _fold_bn(layer["W2"], layer["b2"], layer["bn2"])
        h = _tc_mlp_layer(h, part, w1f, b1f.reshape(1, HID),
                          w2f, b2f.reshape(1, HID), block_rows=2000)
        xs.append(h)

    pad = jnp.zeros((NPAD - N, HID), jnp.float32)
    hp = [jnp.concatenate([hh, pad], axis=0) for hh in xs]
    dump = G + (jnp.arange(NPAD - N, dtype=jnp.int32) % (GPAD - G))
    bidx = jnp.concatenate([batch.astype(jnp.int32), dump])
    pools = _make_sc_pool(NPAD, HID, GPAD)(hp[0], hp[1], hp[2], bidx)

    wl, bl = params["lin1"]
    wl3 = wl.reshape(3, HID, HID)
    return _tc_head(pools[:, :, :G], wl3, bl.reshape(1, HID), G)
